# Initial kernel scaffold; baseline (speedup 1.0000x reference)
import jax
import jax.numpy as jnp
from jax.experimental import pallas as pl


def kernel(feature, edge_index, W1, b1, W2, b2):
    raise NotImplementedError("write your pallas kernel here")


# speedup vs baseline: 3.9876x; 3.9876x over previous
import functools

import jax
import jax.numpy as jnp
from jax import lax
from jax.experimental import pallas as pl
from jax.experimental.pallas import tpu as pltpu
from jax.experimental.pallas import tpu_sc as plsc

N_NODES = 10000
D = 256
HALF = 128
NC = 2
NS = 16
CH = 128
NPAD = 10240
ROWS_PER_TILE = NPAD // NS


def _sc_segment_sum(h_split, src3, dst3, n_chunks):
    mesh = plsc.VectorSubcoreMesh(core_axis_name="c", subcore_axis_name="s")

    @functools.partial(
        pl.kernel,
        mesh=mesh,
        out_type=jax.ShapeDtypeStruct((NC, NPAD, HALF), jnp.float32),
        scratch_types=[
            pltpu.VMEM((n_chunks, CH), jnp.int32),
            pltpu.VMEM((n_chunks, CH), jnp.int32),
            pltpu.VMEM((CH, HALF), jnp.float32),
            pltpu.VMEM_SHARED((NPAD, HALF), jnp.float32),
            pltpu.SemaphoreType.DMA,
        ],
    )
    def agg_kernel(h_hbm, src_hbm, dst_hbm, out_hbm, src_v, dst_v, rows_v,
                   acc, sem):
        c = lax.axis_index("c")
        s = lax.axis_index("s")
        row0 = s * ROWS_PER_TILE

        zero16 = jnp.zeros((16,), jnp.float32)

        def zrow(r, carry):
            for k in range(HALF // 16):
                rows_v[r, pl.ds(k * 16, 16)] = zero16
            return carry

        lax.fori_loop(0, CH, zrow, 0)

        def zcp(j, carry):
            pltpu.sync_copy(rows_v, acc.at[pl.ds(row0 + j * CH, CH)])
            return carry

        lax.fori_loop(0, ROWS_PER_TILE // CH, zcp, 0)
        plsc.subcore_barrier()

        pltpu.sync_copy(src_hbm.at[s], src_v)
        pltpu.sync_copy(dst_hbm.at[s], dst_v)

        table = h_hbm.at[c]

        def chunk(j, carry):
            pltpu.async_copy(table.at[src_v.at[j]], rows_v, sem).wait()
            pltpu.sync_copy(rows_v, acc.at[dst_v.at[j]], add=True)
            return carry

        lax.fori_loop(0, n_chunks, chunk, 0)
        plsc.subcore_barrier()

        pltpu.sync_copy(acc.at[pl.ds(row0, ROWS_PER_TILE)],
                        out_hbm.at[c].at[pl.ds(row0, ROWS_PER_TILE)])

    return agg_kernel(h_split, src3, dst3)


def _tc_linear(agg_split, W, b, relu, split_out):
    BM = 1024
    wr = W.reshape(NC, HALF, D)
    br = b.reshape(1, D)

    def body(a_ref, w_ref, b_ref, o_ref):
        acc = jnp.dot(a_ref[0], w_ref[0], preferred_element_type=jnp.float32)
        acc = acc + jnp.dot(a_ref[1], w_ref[1], preferred_element_type=jnp.float32)
        acc = acc + b_ref[...]
        if relu:
            acc = jnp.maximum(acc, 0.0)
        if split_out:
            o_ref[0] = acc[:, :HALF]
            o_ref[1] = acc[:, HALF:]
        else:
            o_ref[...] = acc

    if split_out:
        out_shape = jax.ShapeDtypeStruct((NC, NPAD, HALF), jnp.float32)
        o_spec = pl.BlockSpec((NC, BM, HALF), lambda i: (0, i, 0))
    else:
        out_shape = jax.ShapeDtypeStruct((NPAD, D), jnp.float32)
        o_spec = pl.BlockSpec((BM, D), lambda i: (i, 0))

    return pl.pallas_call(
        body,
        grid=(NPAD // BM,),
        in_specs=[
            pl.BlockSpec((NC, BM, HALF), lambda i: (0, i, 0)),
            pl.BlockSpec((NC, HALF, D), lambda i: (0, 0, 0)),
            pl.BlockSpec((1, D), lambda i: (0, 0)),
        ],
        out_specs=o_spec,
        out_shape=out_shape,
    )(agg_split, wr, br)


def kernel(feature, edge_index, W1, b1, W2, b2):
    src = edge_index[0].astype(jnp.int32)
    dst = edge_index[1].astype(jnp.int32)
    E = src.shape[0]
    ept = -(-E // NS)
    n_chunks = -(-ept // CH)
    epad = NS * n_chunks * CH
    src3 = jnp.concatenate(
        [src, jnp.zeros((epad - E,), jnp.int32)]).reshape(NS, n_chunks, CH)
    dst3 = jnp.concatenate(
        [dst, jnp.full((epad - E,), N_NODES, jnp.int32)]).reshape(NS, n_chunks, CH)

    feat_pad = jnp.pad(feature, ((0, NPAD - N_NODES), (0, 0)))
    h_split = feat_pad.reshape(NPAD, NC, HALF).transpose(1, 0, 2)

    agg1 = _sc_segment_sum(h_split, src3, dst3, n_chunks)
    h1 = _tc_linear(agg1, W1, b1, relu=True, split_out=True)
    agg2 = _sc_segment_sum(h1, src3, dst3, n_chunks)
    out = _tc_linear(agg2, W2, b2, relu=False, split_out=False)
    return out[:N_NODES]
